# baseline (speedup 1.0000x reference)
import functools

import jax
import jax.numpy as jnp
from jax import lax
from jax.experimental import pallas as pl
from jax.experimental.pallas import tpu as pltpu
from jax.experimental.pallas import tpu_sc as plsc

_SLOT_ROWS = 24
_N_SLOTS = 2


def _make_bcast_kernel(batch, seq, hidden, dtype):
    info = plsc.get_sparse_core_info()
    nw = info.num_cores * info.num_subcores
    rows_per_w = seq // nw

    chunk_rows = []
    r = rows_per_w
    while r > 0:
        c = min(_SLOT_ROWS, r)
        chunk_rows.append(c)
        r -= c
    chunk_off = [sum(chunk_rows[:i]) for i in range(len(chunk_rows))]
    n_chunks = len(chunk_rows)
    n_slots = min(_N_SLOTS, n_chunks)

    mesh = plsc.VectorSubcoreMesh(core_axis_name="c", subcore_axis_name="s")

    @functools.partial(
        pl.kernel,
        mesh=mesh,
        out_type=jax.ShapeDtypeStruct((batch, seq, hidden), dtype),
        scratch_types=[
            pltpu.VMEM((n_slots, _SLOT_ROWS, hidden), dtype),
            pltpu.SemaphoreType.DMA((n_slots,)),
            pltpu.SemaphoreType.DMA((n_slots,)),
        ],
    )
    def k(w_hbm, out_hbm, buf, load_sem, store_sem):
        wid = lax.axis_index("s") * info.num_cores + lax.axis_index("c")
        base = wid * rows_per_w
        rot = wid % batch

        def load(c):
            slot = c % n_slots
            return pltpu.make_async_copy(
                w_hbm.at[pl.ds(base + chunk_off[c], chunk_rows[c])],
                buf.at[slot, pl.ds(0, chunk_rows[c])],
                load_sem.at[slot],
            )

        def stores(c):
            slot = c % n_slots
            return [
                pltpu.make_async_copy(
                    buf.at[slot, pl.ds(0, chunk_rows[c])],
                    out_hbm.at[
                        lax.rem(rot + b, batch), pl.ds(base + chunk_off[c], chunk_rows[c])
                    ],
                    store_sem.at[slot],
                )
                for b in range(batch)
            ]

        all_stores = [stores(c) for c in range(n_chunks)]
        for c in range(n_slots):
            load(c).start()
        for c in range(n_chunks):
            if c >= n_slots:
                for cp in all_stores[c - n_slots]:
                    cp.wait()
                load(c).start()
            load(c).wait()
            for cp in all_stores[c]:
                cp.start()
        for c in range(max(0, n_chunks - n_slots), n_chunks):
            for cp in all_stores[c]:
                cp.wait()

    return k


def kernel(input_ids, pos_emb_weight):
    batch, seq = input_ids.shape
    hidden = pos_emb_weight.shape[1]
    k = _make_bcast_kernel(batch, seq, hidden, pos_emb_weight.dtype)
    return k(pos_emb_weight[:seq])


# speedup vs baseline: 1.0107x; 1.0107x over previous
import functools

import jax
import jax.numpy as jnp
from jax import lax
from jax.experimental import pallas as pl
from jax.experimental.pallas import tpu as pltpu
from jax.experimental.pallas import tpu_sc as plsc

_SLOT_ROWS = 16
_N_SLOTS = 2


def _make_bcast_kernel(batch, seq, hidden, dtype):
    info = plsc.get_sparse_core_info()
    nw = info.num_cores * info.num_subcores
    rows_per_w = seq // nw

    chunk_rows = []
    r = rows_per_w
    while r > 0:
        c = min(_SLOT_ROWS, r)
        chunk_rows.append(c)
        r -= c
    chunk_off = [sum(chunk_rows[:i]) for i in range(len(chunk_rows))]
    n_chunks = len(chunk_rows)
    n_slots = min(_N_SLOTS, n_chunks)

    mesh = plsc.VectorSubcoreMesh(core_axis_name="c", subcore_axis_name="s")

    @functools.partial(
        pl.kernel,
        mesh=mesh,
        out_type=jax.ShapeDtypeStruct((batch, seq, hidden), dtype),
        scratch_types=[
            pltpu.VMEM((n_slots, _SLOT_ROWS, hidden), dtype),
            pltpu.SemaphoreType.DMA((n_slots,)),
            pltpu.SemaphoreType.DMA((n_slots,)),
        ],
    )
    def k(w_hbm, out_hbm, buf, load_sem, store_sem):
        wid = lax.axis_index("s") * info.num_cores + lax.axis_index("c")
        base = wid * rows_per_w
        rot = wid % batch

        def load(c):
            slot = c % n_slots
            return pltpu.make_async_copy(
                w_hbm.at[pl.ds(base + chunk_off[c], chunk_rows[c])],
                buf.at[slot, pl.ds(0, chunk_rows[c])],
                load_sem.at[slot],
            )

        def stores(c):
            slot = c % n_slots
            return [
                pltpu.make_async_copy(
                    buf.at[slot, pl.ds(0, chunk_rows[c])],
                    out_hbm.at[
                        lax.rem(rot + b, batch), pl.ds(base + chunk_off[c], chunk_rows[c])
                    ],
                    store_sem.at[slot],
                )
                for b in range(batch)
            ]

        all_stores = [stores(c) for c in range(n_chunks)]
        for c in range(n_slots):
            load(c).start()
        for c in range(n_chunks):
            if c >= n_slots:
                for cp in all_stores[c - n_slots]:
                    cp.wait()
                load(c).start()
            load(c).wait()
            for cp in all_stores[c]:
                cp.start()
        for c in range(max(0, n_chunks - n_slots), n_chunks):
            for cp in all_stores[c]:
                cp.wait()

    return k


def kernel(input_ids, pos_emb_weight):
    batch, seq = input_ids.shape
    hidden = pos_emb_weight.shape[1]
    k = _make_bcast_kernel(batch, seq, hidden, pos_emb_weight.dtype)
    return k(pos_emb_weight[:seq])
